# initial kernel scaffold (unmeasured)
import functools

import jax
import jax.numpy as jnp
from jax import lax
from jax.experimental import pallas as pl
from jax.experimental.pallas import tpu as pltpu

N_DEV = 8
SQ = 1024
D = 1024
HL = 8
DH = 128
SKV = 1024
SCALE = 0.08838834764831843


def kernel(x, Wq, K_ext, V_ext, Wo):
    def body(x_ref, wq_ref, k_ref, v_ref, wo_ref, out_ref,
             comm_ref, k_vmem, v_vmem, send_sems, recv_sems, kv_sems,
             credit_sem):
        my = lax.axis_index("i")
        left = lax.rem(my + N_DEV - 1, N_DEV)
        right = lax.rem(my + 1, N_DEV)

        barrier_sem = pltpu.get_barrier_semaphore()
        for nbr in (left, right):
            pl.semaphore_signal(
                barrier_sem, inc=1,
                device_id=(nbr,), device_id_type=pl.DeviceIdType.MESH,
            )
        pl.semaphore_wait(barrier_sem, 2)

        comm_ref[0, :D, :] = wq_ref[:, :]
        comm_ref[0, D:, :] = wo_ref[:, :]

        row4 = (lax.broadcasted_iota(jnp.int32, (SQ, SKV), 0) // 64) % 4
        col4 = (lax.broadcasted_iota(jnp.int32, (SQ, SKV), 1) // 64) % 4
        mask = row4 == col4

        xv = x_ref[0, :, :]

        for s in range(N_DEV):
            slot = s % 2
            j = lax.rem(my + 2 * N_DEV - s, N_DEV)

            ck = pltpu.make_async_copy(
                k_ref.at[0, :, pl.ds(j * HL, HL), :], k_vmem, kv_sems.at[0])
            cv = pltpu.make_async_copy(
                v_ref.at[0, :, pl.ds(j * HL, HL), :], v_vmem, kv_sems.at[1])
            ck.start()
            cv.start()

            if s > 0:
                if s > 1:
                    pl.semaphore_wait(credit_sem, 1)
                rdma = pltpu.make_async_remote_copy(
                    src_ref=comm_ref.at[(s - 1) % 2],
                    dst_ref=comm_ref.at[slot],
                    send_sem=send_sems.at[(s - 1) % 2],
                    recv_sem=recv_sems.at[slot],
                    device_id=(right,),
                    device_id_type=pl.DeviceIdType.MESH,
                )
                rdma.start()
                rdma.wait()

            ck.wait()
            cv.wait()

            wq_cur = comm_ref[slot, :D, :]
            q = jnp.dot(xv, wq_cur, preferred_element_type=jnp.float32)

            acc = None
            for h in range(HL):
                q_h = q[:, h * DH:(h + 1) * DH]
                k_h = k_vmem[:, h, :]
                s_h = lax.dot_general(
                    q_h, k_h, (((1,), (1,)), ((), ())),
                    preferred_element_type=jnp.float32) * SCALE
                s_h = jnp.where(mask, s_h, -1e9)
                m = jnp.max(s_h, axis=1, keepdims=True)
                p = jnp.exp(s_h - m)
                w = p / jnp.sum(p, axis=1, keepdims=True)
                ctx = jnp.dot(w, v_vmem[:, h, :],
                              preferred_element_type=jnp.float32)
                wo_h = comm_ref[slot, D + h * DH:D + (h + 1) * DH, :]
                contrib = jnp.dot(ctx, wo_h,
                                  preferred_element_type=jnp.float32)
                acc = contrib if acc is None else acc + contrib

            if s == 0:
                out_ref[0, :, :] = acc
            else:
                out_ref[0, :, :] = out_ref[0, :, :] + acc

            if s <= N_DEV - 3:
                pl.semaphore_signal(
                    credit_sem, inc=1,
                    device_id=(left,), device_id_type=pl.DeviceIdType.MESH,
                )

    return pl.pallas_call(
        body,
        out_shape=jax.ShapeDtypeStruct((1, SQ, D), jnp.float32),
        in_specs=[
            pl.BlockSpec(memory_space=pltpu.VMEM),
            pl.BlockSpec(memory_space=pltpu.VMEM),
            pl.BlockSpec(memory_space=pltpu.ANY),
            pl.BlockSpec(memory_space=pltpu.ANY),
            pl.BlockSpec(memory_space=pltpu.VMEM),
        ],
        out_specs=pl.BlockSpec(memory_space=pltpu.VMEM),
        scratch_shapes=[
            pltpu.VMEM((2, 2 * D, D), jnp.float32),
            pltpu.VMEM((SKV, HL, DH), jnp.float32),
            pltpu.VMEM((SKV, HL, DH), jnp.float32),
            pltpu.SemaphoreType.DMA((2,)),
            pltpu.SemaphoreType.DMA((2,)),
            pltpu.SemaphoreType.DMA((2,)),
            pltpu.SemaphoreType.REGULAR,
        ],
        compiler_params=pltpu.CompilerParams(
            collective_id=0,
            vmem_limit_bytes=128 * 1024 * 1024,
        ),
    )(x, Wq, K_ext, V_ext, Wo)


# baseline (device time: 971524 ns/iter reference)
import jax
import jax.numpy as jnp
from jax import lax
from jax.experimental import pallas as pl
from jax.experimental.pallas import tpu as pltpu

N_DEV = 8
SQ = 1024
D = 1024
HL = 8
DH = 128
SKV = 1024
SCALE = 0.08838834764831843


def kernel(x, Wq, K_ext, V_ext, Wo):
    def body(x_ref, wq_ref, k_ref, v_ref, wo_ref, out_ref,
             comm_ref, q_scr, k_head, v_head, send_sems, recv_sems, kv_sems,
             credit_sem):
        my = lax.axis_index("i")
        left = lax.rem(my + N_DEV - 1, N_DEV)
        right = lax.rem(my + 1, N_DEV)

        barrier_sem = pltpu.get_barrier_semaphore()
        for nbr in (left, right):
            pl.semaphore_signal(
                barrier_sem, inc=1,
                device_id=(nbr,), device_id_type=pl.DeviceIdType.MESH,
            )
        pl.semaphore_wait(barrier_sem, 2)

        comm_ref[0, :D, :] = wq_ref[:, :]
        comm_ref[0, D:, :] = wo_ref[:, :]

        row4 = (lax.broadcasted_iota(jnp.int32, (SQ, SKV), 0) // 64) % 4
        col4 = (lax.broadcasted_iota(jnp.int32, (SQ, SKV), 1) // 64) % 4
        mask = row4 == col4

        out_ref[0, :, :] = jnp.zeros((SQ, D), jnp.float32)
        xv = x_ref[0, :, :]

        for s in range(N_DEV):
            slot = s % 2
            j = lax.rem(my + 2 * N_DEV - s, N_DEV)

            if s > 0:
                if s > 1:
                    pl.semaphore_wait(credit_sem, 1)
                rdma = pltpu.make_async_remote_copy(
                    src_ref=comm_ref.at[(s - 1) % 2],
                    dst_ref=comm_ref.at[slot],
                    send_sem=send_sems.at[(s - 1) % 2],
                    recv_sem=recv_sems.at[slot],
                    device_id=(right,),
                    device_id_type=pl.DeviceIdType.MESH,
                )
                rdma.start()
                rdma.wait()

            wq_cur = comm_ref[slot, :D, :]
            q = jnp.dot(xv, wq_cur, preferred_element_type=jnp.float32)
            for h in range(HL):
                q_scr[h] = q[:, h * DH:(h + 1) * DH]

            def head_step(h, _):
                jj = j * HL + h
                ck = pltpu.make_async_copy(
                    k_ref.at[0, :, jj, :], k_head, kv_sems.at[0])
                cv = pltpu.make_async_copy(
                    v_ref.at[0, :, jj, :], v_head, kv_sems.at[1])
                ck.start()
                cv.start()
                ck.wait()
                cv.wait()
                q_h = q_scr[h]
                s_h = lax.dot_general(
                    q_h, k_head[:, :], (((1,), (1,)), ((), ())),
                    preferred_element_type=jnp.float32) * SCALE
                s_h = jnp.where(mask, s_h, -1e9)
                m = jnp.max(s_h, axis=1, keepdims=True)
                p = jnp.exp(s_h - m)
                w = p / jnp.sum(p, axis=1, keepdims=True)
                ctx = jnp.dot(w, v_head[:, :],
                              preferred_element_type=jnp.float32)
                wo_h = comm_ref[slot, pl.ds(D + h * DH, DH), :]
                out_ref[0, :, :] = out_ref[0, :, :] + jnp.dot(
                    ctx, wo_h, preferred_element_type=jnp.float32)
                return 0

            lax.fori_loop(0, HL, head_step, 0)

            if s <= N_DEV - 3:
                pl.semaphore_signal(
                    credit_sem, inc=1,
                    device_id=(left,), device_id_type=pl.DeviceIdType.MESH,
                )

    return pl.pallas_call(
        body,
        out_shape=jax.ShapeDtypeStruct((1, SQ, D), jnp.float32),
        in_specs=[
            pl.BlockSpec(memory_space=pltpu.VMEM),
            pl.BlockSpec(memory_space=pltpu.VMEM),
            pl.BlockSpec(memory_space=pl.ANY),
            pl.BlockSpec(memory_space=pl.ANY),
            pl.BlockSpec(memory_space=pltpu.VMEM),
        ],
        out_specs=pl.BlockSpec(memory_space=pltpu.VMEM),
        scratch_shapes=[
            pltpu.VMEM((2, 2 * D, D), jnp.float32),
            pltpu.VMEM((HL, SQ, DH), jnp.float32),
            pltpu.VMEM((SKV, DH), jnp.float32),
            pltpu.VMEM((SKV, DH), jnp.float32),
            pltpu.SemaphoreType.DMA((2,)),
            pltpu.SemaphoreType.DMA((2,)),
            pltpu.SemaphoreType.DMA((2,)),
            pltpu.SemaphoreType.REGULAR,
        ],
        compiler_params=pltpu.CompilerParams(
            collective_id=0,
            vmem_limit_bytes=128 * 1024 * 1024,
        ),
    )(x, Wq, K_ext, V_ext, Wo)


# device time: 383548 ns/iter; 2.5330x vs baseline; 2.5330x over previous
import jax
import jax.numpy as jnp
from jax import lax
from jax.experimental import pallas as pl
from jax.experimental.pallas import tpu as pltpu

N_DEV = 8
SQ = 1024
D = 1024
HL = 8
DH = 128
SKV = 1024
HALF = 512
SCALE = 0.08838834764831843


def kernel(x, Wq, K_ext, V_ext, Wo):
    def body(x_ref, wq_ref, k_ref, v_ref, wo_ref, out_ref,
             wq_all, wo_all, q_scr, k_heads, v_heads,
             send_sems, recv_sems, kv_sems, credit_cw, credit_ccw):
        my = lax.axis_index("i")
        left = lax.rem(my + N_DEV - 1, N_DEV)
        right = lax.rem(my + 1, N_DEV)

        barrier_sem = pltpu.get_barrier_semaphore()
        for nbr in (left, right):
            pl.semaphore_signal(
                barrier_sem, inc=1,
                device_id=(nbr,), device_id_type=pl.DeviceIdType.MESH,
            )
        pl.semaphore_wait(barrier_sem, 2)

        wq_all[0, :, :] = wq_ref[:, :]
        wo_all[0, :, :] = wo_ref[:, :]

        row4 = (lax.broadcasted_iota(jnp.int32, (SQ, SKV), 0) // 64) % 4
        col4 = (lax.broadcasted_iota(jnp.int32, (SQ, SKV), 1) // 64) % 4
        mask = row4 == col4

        out_ref[0, :, :] = jnp.zeros((SQ, D), jnp.float32)
        xv = x_ref[0, :, :]

        def make_hops(h):
            src, dst = (h - 1) % 2, h % 2
            hops = []
            for b, (ref, mk, dev) in enumerate((
                (wq_all, lambda r, sl: r.at[sl, :, pl.ds(0, HALF)], right),
                (wo_all, lambda r, sl: r.at[sl, pl.ds(0, HALF), :], right),
                (wq_all, lambda r, sl: r.at[sl, :, pl.ds(HALF, HALF)], left),
                (wo_all, lambda r, sl: r.at[sl, pl.ds(HALF, HALF), :], left),
            )):
                hops.append(pltpu.make_async_remote_copy(
                    src_ref=mk(ref, src),
                    dst_ref=mk(ref, dst),
                    send_sem=send_sems.at[b, src],
                    recv_sem=recv_sems.at[b, dst],
                    device_id=(dev,),
                    device_id_type=pl.DeviceIdType.MESH,
                ))
            return hops

        def compute_step(s):
            slot = s % 2
            jA = lax.rem(my + 2 * N_DEV - s, N_DEV)
            jB = lax.rem(my + s, N_DEV)

            def head_src(h):
                return jnp.where(h < HL // 2, jA, jB) * HL + h

            pltpu.make_async_copy(
                k_ref.at[0, :, head_src(0), :], k_heads.at[0],
                kv_sems.at[0, 0]).start()
            pltpu.make_async_copy(
                v_ref.at[0, :, head_src(0), :], v_heads.at[0],
                kv_sems.at[0, 1]).start()

            q = jnp.dot(xv, wq_all[slot, :, :],
                        preferred_element_type=jnp.float32)
            for h in range(HL):
                q_scr[h] = q[:, h * DH:(h + 1) * DH]

            def head_step(h, _):
                buf = lax.rem(h, 2)
                nbuf = lax.rem(h + 1, 2)

                @pl.when(h + 1 < HL)
                def _():
                    jj2 = head_src(h + 1)
                    pltpu.make_async_copy(
                        k_ref.at[0, :, jj2, :], k_heads.at[nbuf],
                        kv_sems.at[nbuf, 0]).start()
                    pltpu.make_async_copy(
                        v_ref.at[0, :, jj2, :], v_heads.at[nbuf],
                        kv_sems.at[nbuf, 1]).start()

                jj = head_src(h)
                pltpu.make_async_copy(
                    k_ref.at[0, :, jj, :], k_heads.at[buf],
                    kv_sems.at[buf, 0]).wait()
                pltpu.make_async_copy(
                    v_ref.at[0, :, jj, :], v_heads.at[buf],
                    kv_sems.at[buf, 1]).wait()

                q_h = q_scr[h]
                s_h = lax.dot_general(
                    q_h, k_heads[buf, :, :], (((1,), (1,)), ((), ())),
                    preferred_element_type=jnp.float32) * SCALE
                s_h = jnp.where(mask, s_h, -1e9)
                m = jnp.max(s_h, axis=1, keepdims=True)
                p = jnp.exp(s_h - m)
                w = p / jnp.sum(p, axis=1, keepdims=True)
                ctx = jnp.dot(w, v_heads[buf, :, :],
                              preferred_element_type=jnp.float32)
                wo_h = wo_all[slot, pl.ds(h * DH, DH), :]
                out_ref[0, :, :] = out_ref[0, :, :] + jnp.dot(
                    ctx, wo_h, preferred_element_type=jnp.float32)
                return 0

            lax.fori_loop(0, HL, head_step, 0)

        hops = make_hops(1)
        for hp in hops:
            hp.start()
        compute_step(0)

        for s in range(1, N_DEV):
            for hp in hops:
                hp.wait()
            if 1 <= s <= N_DEV - 2:
                pl.semaphore_signal(
                    credit_cw, inc=1,
                    device_id=(left,), device_id_type=pl.DeviceIdType.MESH,
                )
                pl.semaphore_signal(
                    credit_ccw, inc=1,
                    device_id=(right,), device_id_type=pl.DeviceIdType.MESH,
                )
            if s < N_DEV - 1:
                pl.semaphore_wait(credit_cw, 1)
                pl.semaphore_wait(credit_ccw, 1)
                hops = make_hops(s + 1)
                for hp in hops:
                    hp.start()
            compute_step(s)

    return pl.pallas_call(
        body,
        out_shape=jax.ShapeDtypeStruct((1, SQ, D), jnp.float32),
        in_specs=[
            pl.BlockSpec(memory_space=pltpu.VMEM),
            pl.BlockSpec(memory_space=pltpu.VMEM),
            pl.BlockSpec(memory_space=pl.ANY),
            pl.BlockSpec(memory_space=pl.ANY),
            pl.BlockSpec(memory_space=pltpu.VMEM),
        ],
        out_specs=pl.BlockSpec(memory_space=pltpu.VMEM),
        scratch_shapes=[
            pltpu.VMEM((2, D, D), jnp.float32),
            pltpu.VMEM((2, D, D), jnp.float32),
            pltpu.VMEM((HL, SQ, DH), jnp.float32),
            pltpu.VMEM((2, SKV, DH), jnp.float32),
            pltpu.VMEM((2, SKV, DH), jnp.float32),
            pltpu.SemaphoreType.DMA((4, 2)),
            pltpu.SemaphoreType.DMA((4, 2)),
            pltpu.SemaphoreType.DMA((2, 2)),
            pltpu.SemaphoreType.REGULAR,
            pltpu.SemaphoreType.REGULAR,
        ],
        compiler_params=pltpu.CompilerParams(
            collective_id=0,
            vmem_limit_bytes=128 * 1024 * 1024,
        ),
    )(x, Wq, K_ext, V_ext, Wo)


# device time: 378566 ns/iter; 2.5663x vs baseline; 1.0132x over previous
import jax
import jax.numpy as jnp
from jax import lax
from jax.experimental import pallas as pl
from jax.experimental.pallas import tpu as pltpu

N_DEV = 8
SQ = 1024
D = 1024
HL = 8
DH = 128
SKV = 1024
HALF = 512
SCALE = 0.08838834764831843


def kernel(x, Wq, K_ext, V_ext, Wo):
    def body(x_ref, wq_ref, k_ref, v_ref, wo_ref, out_ref,
             wq_all, wo_all, q_scr, k_heads, v_heads, out_g,
             send_sems, recv_sems, kv_sems, credit_cw, credit_ccw):
        my = lax.axis_index("i")
        left = lax.rem(my + N_DEV - 1, N_DEV)
        right = lax.rem(my + 1, N_DEV)

        barrier_sem = pltpu.get_barrier_semaphore()
        for nbr in (left, right):
            pl.semaphore_signal(
                barrier_sem, inc=1,
                device_id=(nbr,), device_id_type=pl.DeviceIdType.MESH,
            )
        pl.semaphore_wait(barrier_sem, 2)

        wq_all[0, :, :] = wq_ref[:, :]
        wo_all[0, :, :] = wo_ref[:, :]

        out_g[:, :] = jnp.zeros((SQ, D), jnp.float32)
        xv = x_ref[0, :, :]

        def make_hops(h):
            src, dst = (h - 1) % 2, h % 2
            hops = []
            for b, (ref, mk, dev) in enumerate((
                (wq_all, lambda r, sl: r.at[sl, :, pl.ds(0, HALF)], right),
                (wo_all, lambda r, sl: r.at[sl, pl.ds(0, HALF), :], right),
                (wq_all, lambda r, sl: r.at[sl, :, pl.ds(HALF, HALF)], left),
                (wo_all, lambda r, sl: r.at[sl, pl.ds(HALF, HALF), :], left),
            )):
                hops.append(pltpu.make_async_remote_copy(
                    src_ref=mk(ref, src),
                    dst_ref=mk(ref, dst),
                    send_sem=send_sems.at[b, src],
                    recv_sem=recv_sems.at[b, dst],
                    device_id=(dev,),
                    device_id_type=pl.DeviceIdType.MESH,
                ))
            return hops

        def compute_step(s):
            slot = s % 2
            jA = lax.rem(my + 2 * N_DEV - s, N_DEV)
            jB = lax.rem(my + s, N_DEV)

            def head_src(h):
                return jnp.where(h < HL // 2, jA, jB) * HL + h

            pltpu.make_async_copy(
                k_ref.at[0, :, head_src(0), :], k_heads.at[0],
                kv_sems.at[0, 0]).start()
            pltpu.make_async_copy(
                v_ref.at[0, :, head_src(0), :], v_heads.at[0],
                kv_sems.at[0, 1]).start()

            q = jnp.dot(xv, wq_all[slot, :, :],
                        preferred_element_type=jnp.float32)
            for h in range(HL):
                q_scr[h] = q[:, h * DH:(h + 1) * DH]

            def head_step(h, _):
                buf = lax.rem(h, 2)
                nbuf = lax.rem(h + 1, 2)

                @pl.when(h + 1 < HL)
                def _():
                    jj2 = head_src(h + 1)
                    pltpu.make_async_copy(
                        k_ref.at[0, :, jj2, :], k_heads.at[nbuf],
                        kv_sems.at[nbuf, 0]).start()
                    pltpu.make_async_copy(
                        v_ref.at[0, :, jj2, :], v_heads.at[nbuf],
                        kv_sems.at[nbuf, 1]).start()

                jj = head_src(h)
                pltpu.make_async_copy(
                    k_ref.at[0, :, jj, :], k_heads.at[buf],
                    kv_sems.at[buf, 0]).wait()
                pltpu.make_async_copy(
                    v_ref.at[0, :, jj, :], v_heads.at[buf],
                    kv_sems.at[buf, 1]).wait()

                q_h = q_scr[h]
                k_h = k_heads[buf, :, :]
                v_h = v_heads[buf, :, :]
                wo_h = wo_all[slot, pl.ds(h * DH, DH), :]

                def grp(val, r):
                    return jnp.concatenate(
                        [val[256 * a + 64 * r:256 * a + 64 * r + 64, :]
                         for a in range(4)], axis=0)

                for r in range(4):
                    qg = grp(q_h, r)
                    kg = grp(k_h, r)
                    vg = grp(v_h, r)
                    sg = lax.dot_general(
                        qg, kg, (((1,), (1,)), ((), ())),
                        preferred_element_type=jnp.float32) * SCALE
                    m = jnp.max(sg, axis=1, keepdims=True)
                    p = jnp.exp(sg - m)
                    w = p / jnp.sum(p, axis=1, keepdims=True)
                    ctx = jnp.dot(w, vg, preferred_element_type=jnp.float32)
                    og = jnp.dot(ctx, wo_h,
                                 preferred_element_type=jnp.float32)
                    out_g[256 * r:256 * (r + 1), :] = (
                        out_g[256 * r:256 * (r + 1), :] + og)
                return 0

            lax.fori_loop(0, HL, head_step, 0)

        hops = make_hops(1)
        for hp in hops:
            hp.start()
        compute_step(0)

        for s in range(1, N_DEV):
            for hp in hops:
                hp.wait()
            if 1 <= s <= N_DEV - 2:
                pl.semaphore_signal(
                    credit_cw, inc=1,
                    device_id=(left,), device_id_type=pl.DeviceIdType.MESH,
                )
                pl.semaphore_signal(
                    credit_ccw, inc=1,
                    device_id=(right,), device_id_type=pl.DeviceIdType.MESH,
                )
            if s < N_DEV - 1:
                pl.semaphore_wait(credit_cw, 1)
                pl.semaphore_wait(credit_ccw, 1)
                hops = make_hops(s + 1)
                for hp in hops:
                    hp.start()
            compute_step(s)

        for r in range(4):
            for a in range(4):
                out_ref[0, 256 * a + 64 * r:256 * a + 64 * r + 64, :] = (
                    out_g[256 * r + 64 * a:256 * r + 64 * a + 64, :])

    return pl.pallas_call(
        body,
        out_shape=jax.ShapeDtypeStruct((1, SQ, D), jnp.float32),
        in_specs=[
            pl.BlockSpec(memory_space=pltpu.VMEM),
            pl.BlockSpec(memory_space=pltpu.VMEM),
            pl.BlockSpec(memory_space=pl.ANY),
            pl.BlockSpec(memory_space=pl.ANY),
            pl.BlockSpec(memory_space=pltpu.VMEM),
        ],
        out_specs=pl.BlockSpec(memory_space=pltpu.VMEM),
        scratch_shapes=[
            pltpu.VMEM((2, D, D), jnp.float32),
            pltpu.VMEM((2, D, D), jnp.float32),
            pltpu.VMEM((HL, SQ, DH), jnp.float32),
            pltpu.VMEM((2, SKV, DH), jnp.float32),
            pltpu.VMEM((2, SKV, DH), jnp.float32),
            pltpu.VMEM((SQ, D), jnp.float32),
            pltpu.SemaphoreType.DMA((4, 2)),
            pltpu.SemaphoreType.DMA((4, 2)),
            pltpu.SemaphoreType.DMA((2, 2)),
            pltpu.SemaphoreType.REGULAR,
            pltpu.SemaphoreType.REGULAR,
        ],
        compiler_params=pltpu.CompilerParams(
            collective_id=0,
            vmem_limit_bytes=128 * 1024 * 1024,
        ),
    )(x, Wq, K_ext, V_ext, Wo)
